# baseline (speedup 1.0000x reference)
import jax
import jax.numpy as jnp
from jax.experimental import pallas as pl
from jax.experimental.pallas import tpu as pltpu

_R = 256


def _chamfer_tc_kernel(x1_ref, x2t_ref, d1_ref, d2_ref):
    ib = pl.program_id(1)
    x1 = x1_ref[0]
    x2 = x2t_ref[0]
    n1 = (x1[:, 0:1] * x1[:, 0:1] + x1[:, 1:2] * x1[:, 1:2]
          + x1[:, 2:3] * x1[:, 2:3])
    n2 = (x2[0:1, :] * x2[0:1, :] + x2[1:2, :] * x2[1:2, :]
          + x2[2:3, :] * x2[2:3, :])
    g = jax.lax.dot_general(
        x1, x2,
        dimension_numbers=(((1,), (0,)), ((), ())),
        preferred_element_type=jnp.float32,
        precision=jax.lax.Precision.HIGHEST,
    )
    d = (n1 + n2) - (g + g)
    d1_ref[0, 0, pl.ds(ib * _R, _R)] = jnp.min(d, axis=1)
    colmin = jnp.min(d, axis=0)

    @pl.when(ib == 0)
    def _():
        d2_ref[0, 0, :] = colmin

    @pl.when(ib != 0)
    def _():
        d2_ref[0, 0, :] = jnp.minimum(d2_ref[0, 0, :], colmin)


def kernel(xyz1, xyz2):
    B, N, _ = xyz1.shape
    M = xyz2.shape[1]
    x2t = jnp.swapaxes(xyz2, 1, 2)
    d1, d2 = pl.pallas_call(
        _chamfer_tc_kernel,
        grid=(B, N // _R),
        in_specs=[
            pl.BlockSpec((1, _R, 3), lambda b, i: (b, i, 0)),
            pl.BlockSpec((1, 3, M), lambda b, i: (b, 0, 0)),
        ],
        out_specs=[
            pl.BlockSpec((1, 1, N), lambda b, i: (b, 0, 0)),
            pl.BlockSpec((1, 1, M), lambda b, i: (b, 0, 0)),
        ],
        out_shape=[
            jax.ShapeDtypeStruct((B, 1, N), jnp.float32),
            jax.ShapeDtypeStruct((B, 1, M), jnp.float32),
        ],
        compiler_params=pltpu.CompilerParams(
            dimension_semantics=("parallel", "arbitrary")),
    )(xyz1, x2t)
    return d1.reshape(B, N), d2.reshape(B, M)


# speedup vs baseline: 2.0435x; 2.0435x over previous
import jax
import jax.numpy as jnp
from jax.experimental import pallas as pl
from jax.experimental.pallas import tpu as pltpu

_R = 256


def _chamfer_tc_kernel(x1_ref, x2t_ref, d1_ref, d2_ref):
    ib = pl.program_id(1)
    x1 = x1_ref[0]
    x2 = x2t_ref[0]
    n1 = (x1[:, 0:1] * x1[:, 0:1] + x1[:, 1:2] * x1[:, 1:2]
          + x1[:, 2:3] * x1[:, 2:3])
    n2 = (x2[0:1, :] * x2[0:1, :] + x2[1:2, :] * x2[1:2, :]
          + x2[2:3, :] * x2[2:3, :])
    ones_r = jnp.ones_like(n1)
    lhs = jnp.concatenate([x1 * (-2.0), n1, ones_r], axis=1)
    ones_m = jnp.ones_like(n2)
    rhs = jnp.concatenate([x2, ones_m, n2], axis=0)
    lhs_hi = lhs.astype(jnp.bfloat16)
    lhs_lo = (lhs - lhs_hi.astype(jnp.float32)).astype(jnp.bfloat16)
    rhs_hi = rhs.astype(jnp.bfloat16)
    rhs_lo = (rhs - rhs_hi.astype(jnp.float32)).astype(jnp.bfloat16)
    lhs_aug = jnp.concatenate([lhs_hi, lhs_hi, lhs_lo], axis=1)
    rhs_aug = jnp.concatenate([rhs_hi, rhs_lo, rhs_hi], axis=0)
    d = jax.lax.dot_general(
        lhs_aug, rhs_aug,
        dimension_numbers=(((1,), (0,)), ((), ())),
        preferred_element_type=jnp.float32,
    )
    d1_ref[0, 0, pl.ds(ib * _R, _R)] = jnp.min(d, axis=1)
    colmin = jnp.min(d, axis=0)

    @pl.when(ib == 0)
    def _():
        d2_ref[0, 0, :] = colmin

    @pl.when(ib != 0)
    def _():
        d2_ref[0, 0, :] = jnp.minimum(d2_ref[0, 0, :], colmin)


def kernel(xyz1, xyz2):
    B, N, _ = xyz1.shape
    M = xyz2.shape[1]
    x2t = jnp.swapaxes(xyz2, 1, 2)
    d1, d2 = pl.pallas_call(
        _chamfer_tc_kernel,
        grid=(B, N // _R),
        in_specs=[
            pl.BlockSpec((1, _R, 3), lambda b, i: (b, i, 0)),
            pl.BlockSpec((1, 3, M), lambda b, i: (b, 0, 0)),
        ],
        out_specs=[
            pl.BlockSpec((1, 1, N), lambda b, i: (b, 0, 0)),
            pl.BlockSpec((1, 1, M), lambda b, i: (b, 0, 0)),
        ],
        out_shape=[
            jax.ShapeDtypeStruct((B, 1, N), jnp.float32),
            jax.ShapeDtypeStruct((B, 1, M), jnp.float32),
        ],
        compiler_params=pltpu.CompilerParams(
            dimension_semantics=("parallel", "arbitrary")),
    )(xyz1, x2t)
    return d1.reshape(B, N), d2.reshape(B, M)
